# Initial kernel scaffold; baseline (speedup 1.0000x reference)
import jax
import jax.numpy as jnp
from jax.experimental import pallas as pl


def kernel(xyz, bond_adj, bond_len, bond_par):
    raise NotImplementedError("write your pallas kernel here")


# speedup vs baseline: 59.0442x; 59.0442x over previous
import functools

import jax
import jax.numpy as jnp
from jax import lax
from jax.experimental import pallas as pl
from jax.experimental.pallas import tpu as pltpu
from jax.experimental.pallas import tpu_sc as plsc

N_NODES = 100000
N_EDGES = 3200000

NC = 2
NS = 16
NW = NC * NS

C = 2048
CHUNKS = 49
EPW = C * CHUNKS
E_PAD = EPW * NW

ACC = 100352
ACC_T = ACC // NS
G = C // 16

_MESH = plsc.VectorSubcoreMesh(
    core_axis_name="c", subcore_axis_name="s", num_cores=NC, num_subcores=NS
)


@functools.partial(
    pl.kernel,
    out_type=jax.ShapeDtypeStruct((NC * ACC,), jnp.float32),
    mesh=_MESH,
    scratch_types=[
        pltpu.VMEM((C,), jnp.int32),
        pltpu.VMEM((C,), jnp.int32),
        pltpu.VMEM((C,), jnp.float32),
        pltpu.VMEM((C,), jnp.float32),
        pltpu.VMEM((C,), jnp.float32),
        pltpu.VMEM((C,), jnp.float32),
        pltpu.VMEM((C,), jnp.float32),
        pltpu.VMEM((C,), jnp.float32),
        pltpu.VMEM((C,), jnp.float32),
        pltpu.VMEM((C,), jnp.float32),
        pltpu.VMEM((C,), jnp.float32),
        pltpu.VMEM((ACC_T,), jnp.float32),
        pltpu.VMEM_SHARED((ACC,), jnp.float32),
        pltpu.VMEM_SHARED((ACC,), jnp.float32),
        pltpu.VMEM_SHARED((ACC,), jnp.float32),
        pltpu.VMEM_SHARED((ACC,), jnp.float32),
        pltpu.SemaphoreType.DMA,
    ],
)
def _bond_energy(idx0_hbm, idx1_hbm, len_hbm, par_hbm, x_hbm, y_hbm, z_hbm,
                 out_hbm,
                 idx0_v, idx1_v, len_v, par_v,
                 ax_v, ay_v, az_v, bx_v, by_v, bz_v, eb_v,
                 zbuf, acc_sh, x_sh, y_sh, z_sh, sem):
    cid = lax.axis_index("c")
    sid = lax.axis_index("s")
    wid = cid * NS + sid
    tslice = pl.ds(sid * ACC_T, ACC_T)

    def _zero(i, carry):
        zbuf[pl.ds(i * 16, 16)] = jnp.zeros((16,), jnp.float32)
        return carry

    lax.fori_loop(0, ACC_T // 16, _zero, 0)
    pltpu.sync_copy(zbuf, acc_sh.at[tslice])

    pltpu.sync_copy(x_hbm.at[tslice], zbuf)
    pltpu.sync_copy(zbuf, x_sh.at[tslice])
    pltpu.sync_copy(y_hbm.at[tslice], zbuf)
    pltpu.sync_copy(zbuf, y_sh.at[tslice])
    pltpu.sync_copy(z_hbm.at[tslice], zbuf)
    pltpu.sync_copy(zbuf, z_sh.at[tslice])
    plsc.subcore_barrier()

    base_w = wid * EPW

    def _chunk(t, carry):
        base = base_w + t * C
        pltpu.sync_copy(idx0_hbm.at[pl.ds(base, C)], idx0_v)
        pltpu.sync_copy(idx1_hbm.at[pl.ds(base, C)], idx1_v)
        pltpu.sync_copy(len_hbm.at[pl.ds(base, C)], len_v)
        pltpu.sync_copy(par_hbm.at[pl.ds(base, C)], par_v)
        cps = [
            pltpu.async_copy(x_sh.at[idx0_v], ax_v, sem),
            pltpu.async_copy(y_sh.at[idx0_v], ay_v, sem),
            pltpu.async_copy(z_sh.at[idx0_v], az_v, sem),
            pltpu.async_copy(x_sh.at[idx1_v], bx_v, sem),
            pltpu.async_copy(y_sh.at[idx1_v], by_v, sem),
            pltpu.async_copy(z_sh.at[idx1_v], bz_v, sem),
        ]
        for cp in cps:
            cp.wait()

        def _group(g, gcarry):
            gs = pl.ds(g * 16, 16)
            dx = ax_v[gs] - bx_v[gs]
            dy = ay_v[gs] - by_v[gs]
            dz = az_v[gs] - bz_v[gs]
            d2 = dx * dx + dy * dy + dz * dz
            d2c = jnp.maximum(d2, jnp.float32(1e-30))
            bits = lax.bitcast_convert_type(d2c, jnp.int32)
            r = lax.bitcast_convert_type(
                jnp.int32(0x5F3759DF) - (bits >> 1), jnp.float32
            )
            r = r * (1.5 - 0.5 * d2c * r * r)
            r = r * (1.5 - 0.5 * d2c * r * r)
            r = r * (1.5 - 0.5 * d2c * r * r)
            e = d2 * r
            diff = e - len_v[gs]
            eb_v[gs] = 0.5 * par_v[gs] * diff * diff
            return gcarry

        lax.fori_loop(0, G, _group, 0)
        pltpu.sync_copy(eb_v, acc_sh.at[idx0_v], add=True)
        return carry

    lax.fori_loop(0, CHUNKS, _chunk, 0)
    plsc.subcore_barrier()

    pltpu.sync_copy(acc_sh.at[tslice], zbuf)
    pltpu.sync_copy(zbuf, out_hbm.at[pl.ds(cid * ACC + sid * ACC_T, ACC_T)])


HALF = ACC_T // 2


@functools.partial(
    pl.kernel,
    out_type=jax.ShapeDtypeStruct((ACC,), jnp.float32),
    mesh=_MESH,
    scratch_types=[
        pltpu.VMEM((HALF,), jnp.float32),
        pltpu.VMEM((HALF,), jnp.float32),
    ],
)
def _combine(p_hbm, out_hbm, a_v, b_v):
    cid = lax.axis_index("c")
    sid = lax.axis_index("s")
    off = sid * ACC_T + cid * HALF
    pltpu.sync_copy(p_hbm.at[pl.ds(off, HALF)], a_v)
    pltpu.sync_copy(p_hbm.at[pl.ds(ACC + off, HALF)], b_v)

    def _add(i, carry):
        a_v[pl.ds(i * 16, 16)] = a_v[pl.ds(i * 16, 16)] + b_v[pl.ds(i * 16, 16)]
        return carry

    lax.fori_loop(0, HALF // 16, _add, 0)
    pltpu.sync_copy(a_v, out_hbm.at[pl.ds(off, HALF)])


def kernel(xyz, bond_adj, bond_len, bond_par):
    idx0 = bond_adj[:, 0].astype(jnp.int32)
    idx1 = bond_adj[:, 1].astype(jnp.int32)
    ln = bond_len[:, 0]
    pr = bond_par[:, 0]
    pad = E_PAD - N_EDGES
    zi = jnp.zeros((pad,), jnp.int32)
    zf = jnp.zeros((pad,), jnp.float32)
    idx0 = jnp.concatenate([idx0, zi])
    idx1 = jnp.concatenate([idx1, zi])
    ln = jnp.concatenate([ln, zf])
    pr = jnp.concatenate([pr, zf])
    npad = ACC - N_NODES
    znf = jnp.zeros((npad,), jnp.float32)
    xp = jnp.concatenate([xyz[:, 0], znf])
    yp = jnp.concatenate([xyz[:, 1], znf])
    zp = jnp.concatenate([xyz[:, 2], znf])
    partials = _bond_energy(idx0, idx1, ln, pr, xp, yp, zp)
    out = _combine(partials)
    return out[:N_NODES][:, None]


# speedup vs baseline: 88.5400x; 1.4996x over previous
import functools

import jax
import jax.numpy as jnp
from jax import lax
from jax.experimental import pallas as pl
from jax.experimental.pallas import tpu as pltpu
from jax.experimental.pallas import tpu_sc as plsc

N_NODES = 100000
N_EDGES = 3200000

NC = 2
NS = 16
NW = NC * NS

C = 2000
CHUNKS = 50
EPW = C * CHUNKS

ACC = 100352
ACC_T = ACC // NS
G = C // 16

_MESH = plsc.VectorSubcoreMesh(
    core_axis_name="c", subcore_axis_name="s", num_cores=NC, num_subcores=NS
)

_EDGE_VMEM = (
    [pltpu.VMEM((C,), jnp.int32)] * 2
    + [pltpu.VMEM((C,), jnp.int32)] * 2
    + [pltpu.VMEM((C,), jnp.float32)] * 18
)


@functools.partial(
    pl.kernel,
    out_type=jax.ShapeDtypeStruct((NC * ACC,), jnp.float32),
    mesh=_MESH,
    scratch_types=_EDGE_VMEM
    + [
        pltpu.VMEM((ACC_T,), jnp.float32),
        pltpu.VMEM_SHARED((ACC,), jnp.float32),
        pltpu.VMEM_SHARED((ACC,), jnp.float32),
        pltpu.VMEM_SHARED((ACC,), jnp.float32),
        pltpu.VMEM_SHARED((ACC,), jnp.float32),
        pltpu.SemaphoreType.DMA,
        pltpu.SemaphoreType.DMA,
    ],
)
def _bond_energy(idx0_hbm, idx1_hbm, len_hbm, par_hbm, x_hbm, y_hbm, z_hbm,
                 out_hbm,
                 idx0_a, idx0_b, idx1_a, idx1_b, len_a, len_b, par_a, par_b,
                 ax_a, ax_b, ay_a, ay_b, az_a, az_b,
                 bx_a, bx_b, by_a, by_b, bz_a, bz_b, eb_a, eb_b,
                 zbuf, acc_sh, x_sh, y_sh, z_sh, sem_l, sem_g):
    idx0 = (idx0_a, idx0_b)
    idx1 = (idx1_a, idx1_b)
    ln = (len_a, len_b)
    pr = (par_a, par_b)
    ax = (ax_a, ax_b)
    ay = (ay_a, ay_b)
    az = (az_a, az_b)
    bx = (bx_a, bx_b)
    by = (by_a, by_b)
    bz = (bz_a, bz_b)
    eb = (eb_a, eb_b)

    cid = lax.axis_index("c")
    sid = lax.axis_index("s")
    wid = cid * NS + sid
    tslice = pl.ds(sid * ACC_T, ACC_T)

    def _zero(i, carry):
        zbuf[pl.ds(i * 16, 16)] = jnp.zeros((16,), jnp.float32)
        return carry

    lax.fori_loop(0, ACC_T // 16, _zero, 0)
    pltpu.sync_copy(zbuf, acc_sh.at[tslice])

    pltpu.sync_copy(x_hbm.at[tslice], zbuf)
    pltpu.sync_copy(zbuf, x_sh.at[tslice])
    pltpu.sync_copy(y_hbm.at[tslice], zbuf)
    pltpu.sync_copy(zbuf, y_sh.at[tslice])
    pltpu.sync_copy(z_hbm.at[tslice], zbuf)
    pltpu.sync_copy(zbuf, z_sh.at[tslice])
    plsc.subcore_barrier()

    base_w = wid * EPW

    def lin_parts(t, b):
        base = base_w + t * C
        sl = pl.ds(base, C)
        return (
            (idx0_hbm.at[sl], idx0[b]),
            (idx1_hbm.at[sl], idx1[b]),
            (len_hbm.at[sl], ln[b]),
            (par_hbm.at[sl], pr[b]),
        )

    def lin_issue(t, b):
        for src, dst in lin_parts(t, b):
            pltpu.async_copy(src, dst, sem_l)

    def lin_wait(t, b):
        for src, dst in lin_parts(t, b):
            pltpu.make_async_copy(src, dst, sem_l).wait()

    def gat_parts(b):
        return (
            (x_sh.at[idx0[b]], ax[b]),
            (y_sh.at[idx0[b]], ay[b]),
            (z_sh.at[idx0[b]], az[b]),
            (x_sh.at[idx1[b]], bx[b]),
            (y_sh.at[idx1[b]], by[b]),
            (z_sh.at[idx1[b]], bz[b]),
        )

    def gat_issue(b):
        for src, dst in gat_parts(b):
            pltpu.async_copy(src, dst, sem_g)

    def gat_wait(b):
        for src, dst in gat_parts(b):
            pltpu.make_async_copy(src, dst, sem_g).wait()

    def compute(b):
        def _group(g, gcarry):
            gs = pl.ds(g * 16, 16)
            dx = ax[b][gs] - bx[b][gs]
            dy = ay[b][gs] - by[b][gs]
            dz = az[b][gs] - bz[b][gs]
            d2 = dx * dx + dy * dy + dz * dz
            d2c = jnp.maximum(d2, jnp.float32(1e-30))
            bits = lax.bitcast_convert_type(d2c, jnp.int32)
            r = lax.bitcast_convert_type(
                jnp.int32(0x5F3759DF) - (bits >> 1), jnp.float32
            )
            r = r * (1.5 - 0.5 * d2c * r * r)
            r = r * (1.5 - 0.5 * d2c * r * r)
            e = d2 * r
            diff = e - ln[b][gs]
            eb[b][gs] = 0.5 * pr[b][gs] * diff * diff
            return gcarry

        lax.fori_loop(0, G, _group, 0)

    def scatter(b):
        pltpu.sync_copy(eb[b], acc_sh.at[idx0[b]], add=True)

    for src, dst in lin_parts(0, 0):
        pltpu.sync_copy(src, dst)
    gat_issue(0)
    lin_issue(1, 1)

    def _two(u, carry):
        t0 = 2 * u
        gat_wait(0)
        lin_wait(t0 + 1, 1)
        gat_issue(1)
        compute(0)
        scatter(0)

        @pl.when(u + 1 < CHUNKS // 2)
        def _():
            lin_issue(t0 + 2, 0)

        gat_wait(1)

        @pl.when(u + 1 < CHUNKS // 2)
        def _():
            lin_wait(t0 + 2, 0)
            gat_issue(0)

        compute(1)
        scatter(1)

        @pl.when(u + 1 < CHUNKS // 2)
        def _():
            lin_issue(t0 + 3, 1)

        return carry

    lax.fori_loop(0, CHUNKS // 2, _two, 0)
    plsc.subcore_barrier()

    pltpu.sync_copy(acc_sh.at[tslice], zbuf)
    pltpu.sync_copy(zbuf, out_hbm.at[pl.ds(cid * ACC + sid * ACC_T, ACC_T)])


HALF = ACC_T // 2


@functools.partial(
    pl.kernel,
    out_type=jax.ShapeDtypeStruct((ACC,), jnp.float32),
    mesh=_MESH,
    scratch_types=[
        pltpu.VMEM((HALF,), jnp.float32),
        pltpu.VMEM((HALF,), jnp.float32),
    ],
)
def _combine(p_hbm, out_hbm, a_v, b_v):
    cid = lax.axis_index("c")
    sid = lax.axis_index("s")
    off = sid * ACC_T + cid * HALF
    pltpu.sync_copy(p_hbm.at[pl.ds(off, HALF)], a_v)
    pltpu.sync_copy(p_hbm.at[pl.ds(ACC + off, HALF)], b_v)

    def _add(i, carry):
        a_v[pl.ds(i * 16, 16)] = a_v[pl.ds(i * 16, 16)] + b_v[pl.ds(i * 16, 16)]
        return carry

    lax.fori_loop(0, HALF // 16, _add, 0)
    pltpu.sync_copy(a_v, out_hbm.at[pl.ds(off, HALF)])


def kernel(xyz, bond_adj, bond_len, bond_par):
    idx0 = bond_adj[:, 0].astype(jnp.int32)
    idx1 = bond_adj[:, 1].astype(jnp.int32)
    ln = bond_len[:, 0]
    pr = bond_par[:, 0]
    npad = ACC - N_NODES
    znf = jnp.zeros((npad,), jnp.float32)
    xp = jnp.concatenate([xyz[:, 0], znf])
    yp = jnp.concatenate([xyz[:, 1], znf])
    zp = jnp.concatenate([xyz[:, 2], znf])
    partials = _bond_energy(idx0, idx1, ln, pr, xp, yp, zp)
    out = _combine(partials)
    return out[:N_NODES][:, None]


# speedup vs baseline: 89.3421x; 1.0091x over previous
import functools

import jax
import jax.numpy as jnp
from jax import lax
from jax.experimental import pallas as pl
from jax.experimental.pallas import tpu as pltpu
from jax.experimental.pallas import tpu_sc as plsc

N_NODES = 100000
N_EDGES = 3200000

NC = 2
NS = 16
NW = NC * NS

C = 2000
CHUNKS = 50
EPW = C * CHUNKS

ACC = 100352
ACC_T = ACC // NS
G = C // 16

_MESH = plsc.VectorSubcoreMesh(
    core_axis_name="c", subcore_axis_name="s", num_cores=NC, num_subcores=NS
)

_EDGE_VMEM = (
    [pltpu.VMEM((C,), jnp.int32)] * 2
    + [pltpu.VMEM((C,), jnp.int32)] * 2
    + [pltpu.VMEM((C,), jnp.int32)] * 2
    + [pltpu.VMEM((C,), jnp.float32)] * 18
)


@functools.partial(
    pl.kernel,
    out_type=jax.ShapeDtypeStruct((NC * ACC,), jnp.float32),
    mesh=_MESH,
    scratch_types=_EDGE_VMEM
    + [
        pltpu.VMEM((ACC_T,), jnp.float32),
        pltpu.VMEM_SHARED((ACC,), jnp.float32),
        pltpu.VMEM_SHARED((ACC,), jnp.float32),
        pltpu.VMEM_SHARED((ACC,), jnp.float32),
        pltpu.VMEM_SHARED((ACC,), jnp.float32),
        pltpu.SemaphoreType.DMA,
        pltpu.SemaphoreType.DMA,
        pltpu.SemaphoreType.DMA,
    ],
)
def _bond_energy(idx0_hbm, idx1_hbm, len_hbm, par_hbm, x_hbm, y_hbm, z_hbm,
                 out_hbm,
                 idx0_a, idx0_b, idx1_a, idx1_b, sidx_a, sidx_b,
                 len_a, len_b, par_a, par_b,
                 ax_a, ax_b, ay_a, ay_b, az_a, az_b,
                 bx_a, bx_b, by_a, by_b, bz_a, bz_b, eb_a, eb_b,
                 zbuf, acc_sh, x_sh, y_sh, z_sh, sem_l, sem_g, sem_s):
    idx0 = (idx0_a, idx0_b)
    idx1 = (idx1_a, idx1_b)
    sidx = (sidx_a, sidx_b)
    ln = (len_a, len_b)
    pr = (par_a, par_b)
    ax = (ax_a, ax_b)
    ay = (ay_a, ay_b)
    az = (az_a, az_b)
    bx = (bx_a, bx_b)
    by = (by_a, by_b)
    bz = (bz_a, bz_b)
    eb = (eb_a, eb_b)

    cid = lax.axis_index("c")
    sid = lax.axis_index("s")
    wid = cid * NS + sid
    tslice = pl.ds(sid * ACC_T, ACC_T)

    def _zero(i, carry):
        zbuf[pl.ds(i * 16, 16)] = jnp.zeros((16,), jnp.float32)
        return carry

    lax.fori_loop(0, ACC_T // 16, _zero, 0)
    pltpu.sync_copy(zbuf, acc_sh.at[tslice])

    pltpu.sync_copy(x_hbm.at[tslice], zbuf)
    pltpu.sync_copy(zbuf, x_sh.at[tslice])
    pltpu.sync_copy(y_hbm.at[tslice], zbuf)
    pltpu.sync_copy(zbuf, y_sh.at[tslice])
    pltpu.sync_copy(z_hbm.at[tslice], zbuf)
    pltpu.sync_copy(zbuf, z_sh.at[tslice])
    plsc.subcore_barrier()

    base_w = wid * EPW

    def lin_parts(t, b):
        base = base_w + t * C
        sl = pl.ds(base, C)
        return (
            (idx0_hbm.at[sl], idx0[b]),
            (idx1_hbm.at[sl], idx1[b]),
            (len_hbm.at[sl], ln[b]),
            (par_hbm.at[sl], pr[b]),
        )

    def lin_issue(t, b):
        for src, dst in lin_parts(t, b):
            pltpu.async_copy(src, dst, sem_l)

    def lin_wait(t, b):
        for src, dst in lin_parts(t, b):
            pltpu.make_async_copy(src, dst, sem_l).wait()

    def gat_parts(b):
        return (
            (x_sh.at[idx0[b]], ax[b]),
            (y_sh.at[idx0[b]], ay[b]),
            (z_sh.at[idx0[b]], az[b]),
            (x_sh.at[idx1[b]], bx[b]),
            (y_sh.at[idx1[b]], by[b]),
            (z_sh.at[idx1[b]], bz[b]),
        )

    def gat_issue(b):
        for src, dst in gat_parts(b):
            pltpu.async_copy(src, dst, sem_g)

    def gat_wait(b):
        for src, dst in gat_parts(b):
            pltpu.make_async_copy(src, dst, sem_g).wait()

    def compute(b):
        def _group(g, gcarry):
            gs = pl.ds(g * 16, 16)
            dx = ax[b][gs] - bx[b][gs]
            dy = ay[b][gs] - by[b][gs]
            dz = az[b][gs] - bz[b][gs]
            d2 = dx * dx + dy * dy + dz * dz
            d2c = jnp.maximum(d2, jnp.float32(1e-30))
            bits = lax.bitcast_convert_type(d2c, jnp.int32)
            r = lax.bitcast_convert_type(
                jnp.int32(0x5F3759DF) - (bits >> 1), jnp.float32
            )
            r = r * (1.5 - 0.5 * d2c * r * r)
            r = r * (1.5 - 0.5 * d2c * r * r)
            r = r * (1.5 - 0.5 * d2c * r * r)
            e = d2 * r
            diff = e - ln[b][gs]
            eb[b][gs] = 0.5 * pr[b][gs] * diff * diff
            sidx[b][gs] = idx0[b][gs]
            return gcarry

        lax.fori_loop(0, G, _group, 0)

    def scat_issue(b):
        pltpu.async_copy(eb[b], acc_sh.at[sidx[b]], sem_s, add=True)

    def scat_wait(b):
        pltpu.make_async_copy(eb[b], acc_sh.at[sidx[b]], sem_s).wait()

    for src, dst in lin_parts(0, 0):
        pltpu.sync_copy(src, dst)
    gat_issue(0)
    lin_issue(1, 1)

    def _two(u, carry):
        t0 = 2 * u
        gat_wait(0)
        lin_wait(t0 + 1, 1)
        gat_issue(1)

        @pl.when(u > 0)
        def _():
            scat_wait(0)

        compute(0)
        scat_issue(0)

        @pl.when(u + 1 < CHUNKS // 2)
        def _():
            lin_issue(t0 + 2, 0)

        gat_wait(1)

        @pl.when(u + 1 < CHUNKS // 2)
        def _():
            lin_wait(t0 + 2, 0)
            gat_issue(0)

        @pl.when(u > 0)
        def _():
            scat_wait(1)

        compute(1)
        scat_issue(1)

        @pl.when(u + 1 < CHUNKS // 2)
        def _():
            lin_issue(t0 + 3, 1)

        return carry

    lax.fori_loop(0, CHUNKS // 2, _two, 0)
    scat_wait(0)
    scat_wait(1)
    plsc.subcore_barrier()

    pltpu.sync_copy(acc_sh.at[tslice], zbuf)
    pltpu.sync_copy(zbuf, out_hbm.at[pl.ds(cid * ACC + sid * ACC_T, ACC_T)])


HALF = ACC_T // 2


@functools.partial(
    pl.kernel,
    out_type=jax.ShapeDtypeStruct((ACC,), jnp.float32),
    mesh=_MESH,
    scratch_types=[
        pltpu.VMEM((HALF,), jnp.float32),
        pltpu.VMEM((HALF,), jnp.float32),
    ],
)
def _combine(p_hbm, out_hbm, a_v, b_v):
    cid = lax.axis_index("c")
    sid = lax.axis_index("s")
    off = sid * ACC_T + cid * HALF
    pltpu.sync_copy(p_hbm.at[pl.ds(off, HALF)], a_v)
    pltpu.sync_copy(p_hbm.at[pl.ds(ACC + off, HALF)], b_v)

    def _add(i, carry):
        a_v[pl.ds(i * 16, 16)] = a_v[pl.ds(i * 16, 16)] + b_v[pl.ds(i * 16, 16)]
        return carry

    lax.fori_loop(0, HALF // 16, _add, 0)
    pltpu.sync_copy(a_v, out_hbm.at[pl.ds(off, HALF)])


def kernel(xyz, bond_adj, bond_len, bond_par):
    idx0 = bond_adj[:, 0].astype(jnp.int32)
    idx1 = bond_adj[:, 1].astype(jnp.int32)
    ln = bond_len[:, 0]
    pr = bond_par[:, 0]
    npad = ACC - N_NODES
    znf = jnp.zeros((npad,), jnp.float32)
    xp = jnp.concatenate([xyz[:, 0], znf])
    yp = jnp.concatenate([xyz[:, 1], znf])
    zp = jnp.concatenate([xyz[:, 2], znf])
    partials = _bond_energy(idx0, idx1, ln, pr, xp, yp, zp)
    out = _combine(partials)
    return out[:N_NODES][:, None]
